# baseline (device time: 8331 ns/iter reference)
import jax
import jax.numpy as jnp
from jax import lax
from jax.experimental import pallas as pl
from jax.experimental.pallas import tpu as pltpu


def kernel(x):
    m, n2 = x.shape
    n = n2 // 2
    M = 2 * m

    def body(x_ref, out_ref, send_sem, recv_sem, local_sem):
        mx = lax.axis_index("x")
        my = lax.axis_index("y")

        barrier_sem = pltpu.get_barrier_semaphore()
        pl.semaphore_signal(
            barrier_sem,
            inc=1,
            device_id=(1 - mx, my),
            device_id_type=pl.DeviceIdType.MESH,
        )
        pl.semaphore_wait(barrier_sem, 1)

        rdma = pltpu.make_async_remote_copy(
            src_ref=x_ref.at[:, pl.ds((1 - mx) * n, n)],
            dst_ref=out_ref.at[pl.ds(mx * m, m), :],
            send_sem=send_sem,
            recv_sem=recv_sem,
            device_id=(1 - mx, my),
            device_id_type=pl.DeviceIdType.MESH,
        )
        rdma.start()

        local = pltpu.make_async_copy(
            x_ref.at[:, pl.ds(mx * n, n)],
            out_ref.at[pl.ds(mx * m, m), :],
            local_sem,
        )
        local.start()
        local.wait()

        rdma.wait()

    out_shape = jax.ShapeDtypeStruct((M, n), x.dtype)
    return pl.pallas_call(
        body,
        out_shape=out_shape,
        in_specs=[pl.BlockSpec(memory_space=pltpu.VMEM)],
        out_specs=pl.BlockSpec(memory_space=pl.ANY),
        scratch_shapes=[
            pltpu.SemaphoreType.DMA,
            pltpu.SemaphoreType.DMA,
            pltpu.SemaphoreType.DMA,
        ],
        compiler_params=pltpu.CompilerParams(collective_id=0),
    )(x)
